# Initial kernel scaffold; baseline (speedup 1.0000x reference)
import jax
import jax.numpy as jnp
from jax.experimental import pallas as pl


def kernel(x, table):
    raise NotImplementedError("write your pallas kernel here")


# speedup vs baseline: 1.6157x; 1.6157x over previous
import functools

import jax
import jax.numpy as jnp
from jax import lax
from jax.experimental import pallas as pl
from jax.experimental.pallas import tpu as pltpu
from jax.experimental.pallas import tpu_sc as plsc

B = 16384
L = 20
EMB = 32
B_TOT = B * L

_info = plsc.get_sparse_core_info()
_NC = _info.num_cores
_NS = _info.num_subcores
NW = _NC * _NS
B_PER_W = B_TOT // NW
CHUNK = 2048
N_STEPS = B_PER_W // CHUNK

_mesh = plsc.VectorSubcoreMesh(core_axis_name="c", subcore_axis_name="s")


@functools.partial(
    pl.kernel,
    mesh=_mesh,
    out_type=jax.ShapeDtypeStruct((B_TOT, EMB), jnp.float32),
    scratch_types=[
        pltpu.VMEM((CHUNK,), jnp.int32),
        pltpu.VMEM((CHUNK, EMB), jnp.float32),
        pltpu.SemaphoreType.DMA,
    ],
    compiler_params=pltpu.CompilerParams(use_tc_tiling_on_sc=False),
)
def _gather_kernel(idx_hbm, table_hbm, out_hbm, idx_v, rows_v, sem):
    wid = lax.axis_index("s") * _NC + lax.axis_index("c")
    base = pl.multiple_of(wid * B_PER_W, CHUNK)

    def step(i, carry):
        off = pl.multiple_of(base + i * CHUNK, CHUNK)
        pltpu.sync_copy(idx_hbm.at[pl.ds(off, CHUNK)], idx_v)
        pltpu.async_copy(table_hbm.at[idx_v], rows_v, sem).wait()
        pltpu.sync_copy(rows_v, out_hbm.at[pl.ds(off, CHUNK)])
        return carry

    lax.fori_loop(0, N_STEPS, step, 0)


def kernel(x, table):
    idx = x.reshape(B_TOT)
    out = _gather_kernel(idx, table)
    return out.reshape(B, L, EMB)


# speedup vs baseline: 1.6229x; 1.0044x over previous
import functools

import jax
import jax.numpy as jnp
from jax import lax
from jax.experimental import pallas as pl
from jax.experimental.pallas import tpu as pltpu
from jax.experimental.pallas import tpu_sc as plsc

B = 16384
L = 20
EMB = 32
B_TOT = B * L

_info = plsc.get_sparse_core_info()
_NC = _info.num_cores
_NS = _info.num_subcores
NW = _NC * _NS
B_PER_W = B_TOT // NW
CHUNK = 1280
N_STEPS = B_PER_W // CHUNK
N_BUF = 2

_mesh = plsc.VectorSubcoreMesh(core_axis_name="c", subcore_axis_name="s")


@functools.partial(
    pl.kernel,
    mesh=_mesh,
    out_type=jax.ShapeDtypeStruct((B_TOT, EMB), jnp.float32),
    scratch_types=[
        pltpu.VMEM((N_STEPS, CHUNK), jnp.int32),
        pltpu.VMEM((N_BUF, CHUNK, EMB), jnp.float32),
        pltpu.SemaphoreType.DMA((N_BUF,)),
        pltpu.SemaphoreType.DMA((N_BUF,)),
    ],
    compiler_params=pltpu.CompilerParams(use_tc_tiling_on_sc=False),
)
def _gather_kernel(idx_hbm, table_hbm, out_hbm, idx_v, rows_v, gsem, osem):
    wid = lax.axis_index("s") * _NC + lax.axis_index("c")
    base = pl.multiple_of(wid * B_PER_W, CHUNK)

    pltpu.sync_copy(idx_hbm.at[wid], idx_v)

    gathers = [None] * N_BUF
    stores = [None] * N_BUF
    gathers[0] = pltpu.async_copy(
        table_hbm.at[idx_v.at[0]], rows_v.at[0], gsem.at[0])
    for g in range(N_STEPS):
        b = g % N_BUF
        if g + 1 < N_STEPS:
            nb = (g + 1) % N_BUF
            if stores[nb] is not None:
                stores[nb].wait()
            gathers[nb] = pltpu.async_copy(
                table_hbm.at[idx_v.at[g + 1]], rows_v.at[nb], gsem.at[nb])
        gathers[b].wait()
        stores[b] = pltpu.async_copy(
            rows_v.at[b],
            out_hbm.at[pl.ds(base + g * CHUNK, CHUNK)],
            osem.at[b])
    for s in stores:
        if s is not None:
            s.wait()


def kernel(x, table):
    idx = x.reshape(NW, N_STEPS, CHUNK)
    out = _gather_kernel(idx, table)
    return out.reshape(B, L, EMB)
